# baseline (speedup 1.0000x reference)
import functools

import jax
import jax.numpy as jnp
from jax import lax
from jax.experimental import pallas as pl
from jax.experimental.pallas import tpu as pltpu
from jax.experimental.pallas import tpu_sc as plsc

NUM_CLASS = 100000
B = 1024
CTX_DIM = 512
N_CLS_CTX = 4
SEQ_LEN = 77
D = N_CLS_CTX * CTX_DIM

_BB = 32


def _sc_gather(label, table):
    info = plsc.get_sparse_core_info()
    nw = info.num_cores * info.num_subcores
    b_per_w = B // nw
    mesh = plsc.VectorSubcoreMesh(core_axis_name="c", subcore_axis_name="s")

    @functools.partial(
        pl.kernel,
        mesh=mesh,
        out_type=jax.ShapeDtypeStruct((B * D,), jnp.float32),
        scratch_types=[
            pltpu.VMEM((b_per_w,), jnp.int32),
            pltpu.VMEM((b_per_w, N_CLS_CTX, CTX_DIM), jnp.float32),
            pltpu.SemaphoreType.DMA,
            pltpu.SemaphoreType.DMA,
        ],
    )
    def gather_kernel(idx_hbm, table_hbm, out_hbm, idx_v, rows_v, sem, sem2):
        wid = lax.axis_index("s") * info.num_cores + lax.axis_index("c")
        base = wid * b_per_w
        pltpu.sync_copy(idx_hbm.at[pl.ds(base, b_per_w)], idx_v)
        pltpu.async_copy(table_hbm.at[idx_v], rows_v, sem).wait()
        descs = []
        for j in range(b_per_w):
            for t in range(N_CLS_CTX):
                off = ((base + j) * N_CLS_CTX + t) * CTX_DIM
                c = pltpu.make_async_copy(
                    rows_v.at[j, t], out_hbm.at[pl.ds(off, CTX_DIM)], sem2)
                c.start()
                descs.append(c)
        for c in descs:
            c.wait()

    return gather_kernel(label, table)


def _tc_assemble_body(cls_hbm, pre_ref, mid_ref, sp_ref, suf_ref, out_ref,
                      cls_v, sem):
    i = pl.program_id(0)
    cp = pltpu.make_async_copy(
        cls_hbm.at[pl.ds(i * _BB * D, _BB * D)], cls_v, sem)
    cp.start()

    def bcast(ref):
        return jnp.broadcast_to(ref[...][:, None, :],
                                (ref.shape[0], _BB, CTX_DIM))

    @pl.when(i < 2)
    def _init():
        out_ref[0:5] = bcast(pre_ref)
        out_ref[9:11] = bcast(mid_ref)
        out_ref[11:15] = bcast(sp_ref)
        out_ref[15:SEQ_LEN] = bcast(suf_ref)
    cp.wait()
    for j in range(_BB):
        for t in range(N_CLS_CTX):
            out_ref[5 + t, j, :] = cls_v[pl.ds((j * N_CLS_CTX + t) * CTX_DIM,
                                               CTX_DIM)]


def kernel(label, cls_ctx, token_prefix, token_middle, token_suffix,
           suffix_prompt):
    cls_flat = _sc_gather(label.astype(jnp.int32), cls_ctx)

    pre = token_prefix[0]
    mid = token_middle[0]
    sp = suffix_prompt[0]
    suf = token_suffix[0]
    suffix_len = suf.shape[0]

    out_tm = pl.pallas_call(
        _tc_assemble_body,
        grid=(B // _BB,),
        in_specs=[
            pl.BlockSpec(memory_space=pl.ANY),
            pl.BlockSpec((5, CTX_DIM), lambda i: (0, 0)),
            pl.BlockSpec((2, CTX_DIM), lambda i: (0, 0)),
            pl.BlockSpec((N_CLS_CTX, CTX_DIM), lambda i: (0, 0)),
            pl.BlockSpec((suffix_len, CTX_DIM), lambda i: (0, 0)),
        ],
        out_specs=pl.BlockSpec((SEQ_LEN, _BB, CTX_DIM), lambda i: (0, i, 0)),
        out_shape=jax.ShapeDtypeStruct((SEQ_LEN, B, CTX_DIM), jnp.float32),
        scratch_shapes=[
            pltpu.VMEM((_BB * D,), jnp.float32),
            pltpu.SemaphoreType.DMA,
        ],
    )(cls_flat, pre, mid, sp, suf)
    return jnp.transpose(out_tm, (1, 0, 2))


# speedup vs baseline: 1.1733x; 1.1733x over previous
import functools

import jax
import jax.numpy as jnp
from jax import lax
from jax.experimental import pallas as pl
from jax.experimental.pallas import tpu as pltpu
from jax.experimental.pallas import tpu_sc as plsc

NUM_CLASS = 100000
B = 1024
CTX_DIM = 512
N_CLS_CTX = 4
SEQ_LEN = 77
D = N_CLS_CTX * CTX_DIM

_BBA = 64
_BBB = 256


def _sc_gather(label, table):
    info = plsc.get_sparse_core_info()
    nw = info.num_cores * info.num_subcores
    b_per_w = B // nw
    mesh = plsc.VectorSubcoreMesh(core_axis_name="c", subcore_axis_name="s")

    @functools.partial(
        pl.kernel,
        mesh=mesh,
        out_type=jax.ShapeDtypeStruct((N_CLS_CTX * B * CTX_DIM,),
                                      jnp.float32),
        scratch_types=[
            pltpu.VMEM((b_per_w,), jnp.int32),
            pltpu.VMEM((b_per_w, N_CLS_CTX, CTX_DIM), jnp.float32),
            pltpu.SemaphoreType.DMA,
            pltpu.SemaphoreType.DMA,
        ],
    )
    def gather_kernel(idx_hbm, table_hbm, out_hbm, idx_v, rows_v, sem, sem2):
        wid = lax.axis_index("s") * info.num_cores + lax.axis_index("c")
        base = wid * b_per_w
        pltpu.sync_copy(idx_hbm.at[pl.ds(base, b_per_w)], idx_v)
        pltpu.async_copy(table_hbm.at[idx_v], rows_v, sem).wait()
        descs = []
        for j in range(b_per_w):
            for t in range(N_CLS_CTX):
                off = (t * B + base + j) * CTX_DIM
                c = pltpu.make_async_copy(
                    rows_v.at[j, t], out_hbm.at[pl.ds(off, CTX_DIM)], sem2)
                c.start()
                descs.append(c)
        for c in descs:
            c.wait()

    return gather_kernel(label, table)


def _fill_body(pre_ref, mid_ref, sp_ref, suf_ref, out_ref):
    i = pl.program_id(0)

    def bcast(ref):
        return jnp.broadcast_to(ref[0][:, None, :],
                                (ref.shape[1], _BBA, CTX_DIM))

    @pl.when(i < 2)
    def _init():
        out_ref[0:5] = bcast(pre_ref)
        out_ref[9:11] = bcast(mid_ref)
        out_ref[11:15] = bcast(sp_ref)
        out_ref[15:SEQ_LEN] = bcast(suf_ref)


def _merge_body(out_in_ref, cls_ref, out_ref):
    del out_in_ref
    for jj in range(_BBB):
        out_ref[0, jj, :] = cls_ref[pl.ds(jj * CTX_DIM, CTX_DIM)]


def kernel(label, cls_ctx, token_prefix, token_middle, token_suffix,
           suffix_prompt):
    cls_flat = _sc_gather(label.astype(jnp.int32), cls_ctx)

    suffix_len = token_suffix.shape[1]
    base_tm = pl.pallas_call(
        _fill_body,
        grid=(B // _BBA,),
        in_specs=[
            pl.BlockSpec((1, 5, CTX_DIM), lambda i: (0, 0, 0)),
            pl.BlockSpec((1, 2, CTX_DIM), lambda i: (0, 0, 0)),
            pl.BlockSpec((1, N_CLS_CTX, CTX_DIM), lambda i: (0, 0, 0)),
            pl.BlockSpec((1, suffix_len, CTX_DIM), lambda i: (0, 0, 0)),
        ],
        out_specs=pl.BlockSpec((SEQ_LEN, _BBA, CTX_DIM), lambda i: (0, i, 0)),
        out_shape=jax.ShapeDtypeStruct((SEQ_LEN, B, CTX_DIM), jnp.float32),
    )(token_prefix, token_middle, suffix_prompt, token_suffix)

    out_tm = pl.pallas_call(
        _merge_body,
        grid=(N_CLS_CTX, B // _BBB),
        in_specs=[
            pl.BlockSpec(memory_space=pl.ANY),
            pl.BlockSpec((_BBB * CTX_DIM,),
                         lambda t, j: (t * (B // _BBB) + j,)),
        ],
        out_specs=pl.BlockSpec((1, _BBB, CTX_DIM),
                               lambda t, j: (5 + t, j, 0)),
        out_shape=jax.ShapeDtypeStruct((SEQ_LEN, B, CTX_DIM), jnp.float32),
        input_output_aliases={0: 0},
    )(base_tm, cls_flat)
    return jnp.transpose(out_tm, (1, 0, 2))


# speedup vs baseline: 1.2639x; 1.0772x over previous
import functools

import jax
import jax.numpy as jnp
from jax import lax
from jax.experimental import pallas as pl
from jax.experimental.pallas import tpu as pltpu
from jax.experimental.pallas import tpu_sc as plsc

NUM_CLASS = 100000
B = 1024
CTX_DIM = 512
N_CLS_CTX = 4
SEQ_LEN = 77
D = N_CLS_CTX * CTX_DIM

_BBA = 64
_BBB = B


def _sc_gather(label, table):
    info = plsc.get_sparse_core_info()
    nw = info.num_cores * info.num_subcores
    b_per_w = B // nw
    mesh = plsc.VectorSubcoreMesh(core_axis_name="c", subcore_axis_name="s")

    @functools.partial(
        pl.kernel,
        mesh=mesh,
        out_type=jax.ShapeDtypeStruct((N_CLS_CTX * B * CTX_DIM,),
                                      jnp.float32),
        scratch_types=[
            pltpu.VMEM((b_per_w,), jnp.int32),
            pltpu.VMEM((b_per_w, N_CLS_CTX, CTX_DIM), jnp.float32),
            pltpu.SemaphoreType.DMA,
            pltpu.SemaphoreType.DMA,
        ],
    )
    def gather_kernel(idx_hbm, table_hbm, out_hbm, idx_v, rows_v, sem, sem2):
        wid = lax.axis_index("s") * info.num_cores + lax.axis_index("c")
        base = wid * b_per_w
        pltpu.sync_copy(idx_hbm.at[pl.ds(base, b_per_w)], idx_v)
        pltpu.async_copy(table_hbm.at[idx_v], rows_v, sem).wait()
        descs = []
        for j in range(b_per_w):
            for t in range(N_CLS_CTX):
                off = (t * B + base + j) * CTX_DIM
                c = pltpu.make_async_copy(
                    rows_v.at[j, t], out_hbm.at[pl.ds(off, CTX_DIM)], sem2)
                c.start()
                descs.append(c)
        for c in descs:
            c.wait()

    return gather_kernel(label, table)


def _fill_body(pre_ref, mid_ref, sp_ref, suf_ref, out_ref):
    i = pl.program_id(0)

    def bcast(ref):
        return jnp.broadcast_to(ref[0][:, None, :],
                                (ref.shape[1], _BBA, CTX_DIM))

    @pl.when(i < 2)
    def _init():
        out_ref[0:5] = bcast(pre_ref)
        out_ref[9:11] = bcast(mid_ref)
        out_ref[11:15] = bcast(sp_ref)
        out_ref[15:SEQ_LEN] = bcast(suf_ref)


def _merge_body(out_in_ref, cls_ref, out_ref):
    del out_in_ref
    out_ref[0] = cls_ref[...].reshape(_BBB, CTX_DIM)


def kernel(label, cls_ctx, token_prefix, token_middle, token_suffix,
           suffix_prompt):
    cls_flat = _sc_gather(label.astype(jnp.int32), cls_ctx)

    suffix_len = token_suffix.shape[1]
    base_tm = pl.pallas_call(
        _fill_body,
        grid=(B // _BBA,),
        in_specs=[
            pl.BlockSpec((1, 5, CTX_DIM), lambda i: (0, 0, 0)),
            pl.BlockSpec((1, 2, CTX_DIM), lambda i: (0, 0, 0)),
            pl.BlockSpec((1, N_CLS_CTX, CTX_DIM), lambda i: (0, 0, 0)),
            pl.BlockSpec((1, suffix_len, CTX_DIM), lambda i: (0, 0, 0)),
        ],
        out_specs=pl.BlockSpec((SEQ_LEN, _BBA, CTX_DIM), lambda i: (0, i, 0)),
        out_shape=jax.ShapeDtypeStruct((SEQ_LEN, B, CTX_DIM), jnp.float32),
    )(token_prefix, token_middle, suffix_prompt, token_suffix)

    out_tm = pl.pallas_call(
        _merge_body,
        grid=(N_CLS_CTX,),
        in_specs=[
            pl.BlockSpec(memory_space=pl.ANY),
            pl.BlockSpec((_BBB * CTX_DIM,), lambda t: (t,)),
        ],
        out_specs=pl.BlockSpec((1, _BBB, CTX_DIM), lambda t: (5 + t, 0, 0)),
        out_shape=jax.ShapeDtypeStruct((SEQ_LEN, B, CTX_DIM), jnp.float32),
        input_output_aliases={0: 0},
    )(base_tm, cls_flat)
    return jnp.transpose(out_tm, (1, 0, 2))
